# baseline (speedup 1.0000x reference)
import functools

import jax
import jax.numpy as jnp
from jax import lax
from jax.experimental import pallas as pl
from jax.experimental.pallas import tpu as pltpu
from jax.experimental.pallas import tpu_sc as plsc

N_NODES = 10000
N_EDGES = 320000
D_FEAT = 128
D_EDGE = 16
D_OUT = 128

NC = 2
NS = 16
NW = NC * NS
C = 64
CH0 = 88
CH1 = 232
TOT_CHUNKS = NS * (CH0 + CH1)
E_PAD = TOT_CHUNKS * C
PK_HALF = 58
N_PAD = 10112
ROWS_PER_TILE = N_PAD // NS
D_DEG = 8
L = 16


def _sc_segment_sums(x, packed2d, ea3d, zx, ze, zd, ones):
  mesh = plsc.VectorSubcoreMesh(core_axis_name="c", subcore_axis_name="s")

  @functools.partial(
      pl.kernel,
      mesh=mesh,
      compiler_params=pltpu.CompilerParams(use_tc_tiling_on_sc=False),
      out_type=[
          jax.ShapeDtypeStruct((NC * N_PAD, D_FEAT), jnp.float32),
          jax.ShapeDtypeStruct((NC * N_PAD, D_EDGE), jnp.float32),
          jax.ShapeDtypeStruct((NC * N_PAD, D_DEG), jnp.float32),
      ],
      scratch_types=[
          pltpu.VMEM_SHARED((N_PAD, D_FEAT), jnp.float32),
          pltpu.VMEM_SHARED((N_PAD, D_EDGE), jnp.float32),
          pltpu.VMEM_SHARED((N_PAD, D_DEG), jnp.float32),
          pltpu.VMEM((PK_HALF, 128), jnp.int32),
          pltpu.VMEM((4, C), jnp.int32),
          pltpu.VMEM((4, C), jnp.int32),
          pltpu.VMEM((C, D_FEAT), jnp.float32),
          pltpu.VMEM((C, D_FEAT), jnp.float32),
          pltpu.VMEM((8, 128), jnp.float32),
          pltpu.VMEM((8, 128), jnp.float32),
          pltpu.VMEM((C, D_EDGE), jnp.float32),
          pltpu.VMEM((C, D_EDGE), jnp.float32),
          pltpu.VMEM((C, D_DEG), jnp.float32),
          pltpu.SemaphoreType.DMA,
          pltpu.SemaphoreType.DMA,
          pltpu.SemaphoreType.DMA,
          pltpu.SemaphoreType.DMA,
          pltpu.SemaphoreType.DMA,
          pltpu.SemaphoreType.DMA,
          pltpu.SemaphoreType.DMA,
          pltpu.SemaphoreType.DMA,
          pltpu.SemaphoreType.DMA,
          pltpu.SemaphoreType.DMA,
      ],
  )
  def k(x_hbm, pk_hbm, ea_hbm, zx_hbm, ze_hbm, zd_hbm, ones_hbm,
        sx_hbm, se_hbm, sd_hbm,
        acc_x, acc_e, acc_d, pk_v, src_i, dst_i,
        rows0, rows1, eaw0, eaw1, ea0, ea1, ones_v,
        sg0, sg1, se0, se1, ssx0, ssx1, sse0, sse1, ssd0, ssd1):
    cid = lax.axis_index("c")
    sid = lax.axis_index("s")
    nch = jnp.where(cid == 0, CH0, CH1)
    cbase = jnp.where(cid == 0, sid * CH0, NS * CH0 + sid * CH1)
    rows = (rows0, rows1)
    eaw = (eaw0, eaw1)
    eab = (ea0, ea1)
    sg = (sg0, sg1)
    sea = (se0, se1)
    ssx = (ssx0, ssx1)
    sse = (sse0, sse1)
    ssd = (ssd0, ssd1)

    r0 = sid * ROWS_PER_TILE
    half = ROWS_PER_TILE // 2
    for t in range(2):
      pltpu.sync_copy(zx_hbm, acc_x.at[pl.ds(r0 + t * half, half)])
    pltpu.sync_copy(ze_hbm, acc_e.at[pl.ds(r0, ROWS_PER_TILE)])
    pltpu.sync_copy(zd_hbm, acc_d.at[pl.ds(r0, ROWS_PER_TILE)])

    @pl.when(cid == 0)
    def _():
      pltpu.sync_copy(pk_hbm.at[pl.ds(cbase // 2, CH0 // 2)],
                      pk_v.at[pl.ds(0, CH0 // 2)])

    @pl.when(cid == 1)
    def _():
      pltpu.sync_copy(pk_hbm.at[pl.ds(cbase // 2, PK_HALF)],
                      pk_v.at[pl.ds(0, PK_HALF)])
    pltpu.sync_copy(ones_hbm, ones_v)
    plsc.subcore_barrier()

    def unpack(row, col0, slot):
      for kk in range(C // L):
        pk = pk_v[row, pl.ds(col0 + kk * L, L)]
        src_i[slot, pl.ds(kk * L, L)] = pk & 0xFFFF
        dst_i[slot, pl.ds(kk * L, L)] = lax.shift_right_logical(pk, 16)

    def ea_row(n):
      g = cbase + n
      return jnp.where(g < N_EDGES // C, g * 8, 0)

    def issue_gather(n, p, slot):
      pltpu.async_copy(x_hbm.at[src_i.at[slot]], rows[p], sg[p])
      pltpu.async_copy(ea_hbm.at[pl.ds(ea_row(n), 8)], eaw[p], sea[p])

    def wait_and_scatter(n, p, slot):
      pltpu.make_async_copy(x_hbm.at[src_i.at[slot]], rows[p], sg[p]).wait()
      pltpu.make_async_copy(ea_hbm.at[pl.ds(ea_row(n), 8)], eaw[p],
                            sea[p]).wait()
      for r in range(C):
        eab[p][r, pl.ds(0, L)] = eaw[p][r // 8, pl.ds((r % 8) * L, L)]
      pltpu.async_copy(rows[p], acc_x.at[dst_i.at[slot]], ssx[p], add=True)
      pltpu.async_copy(eab[p], acc_e.at[dst_i.at[slot]], sse[p], add=True)
      pltpu.async_copy(ones_v, acc_d.at[dst_i.at[slot]], ssd[p], add=True)

    def drain_scatter(p, slot):
      pltpu.make_async_copy(rows[p], acc_x.at[dst_i.at[slot]], ssx[p]).wait()
      pltpu.make_async_copy(eab[p], acc_e.at[dst_i.at[slot]], sse[p]).wait()
      pltpu.make_async_copy(ones_v, acc_d.at[dst_i.at[slot]], ssd[p]).wait()

    for q in range(4):
      unpack(q // 2, 64 * (q % 2), q)
    issue_gather(0, 0, 0)
    issue_gather(1, 1, 1)

    @pl.loop(0, nch, step=4)
    def _body(j):
      @pl.when(j + 4 == 2 * PK_HALF)
      def _():
        pltpu.sync_copy(pk_hbm.at[pl.ds(cbase // 2 + PK_HALF, PK_HALF)],
                        pk_v.at[pl.ds(0, PK_HALF)])

      jrow = j // 2 - jnp.where(j + 4 >= 2 * PK_HALF, PK_HALF, 0)
      wait_and_scatter(j, 0, 0)
      wait_and_scatter(j + 1, 1, 1)
      drain_scatter(0, 0)

      @pl.when(j + 4 < nch)
      def _():
        unpack(jrow + 2, 0, 0)
      issue_gather(j + 2, 0, 2)

      drain_scatter(1, 1)

      @pl.when(j + 4 < nch)
      def _():
        unpack(jrow + 2, 64, 1)
      issue_gather(j + 3, 1, 3)

      wait_and_scatter(j + 2, 0, 2)
      wait_and_scatter(j + 3, 1, 3)

      drain_scatter(0, 2)

      @pl.when(j + 4 < nch)
      def _():
        unpack(jrow + 3, 0, 2)
        issue_gather(j + 4, 0, 0)

      drain_scatter(1, 3)

      @pl.when(j + 4 < nch)
      def _():
        unpack(jrow + 3, 64, 3)
        issue_gather(j + 5, 1, 1)

    plsc.subcore_barrier()

    out0 = cid * N_PAD + r0
    pltpu.sync_copy(acc_x.at[pl.ds(r0, ROWS_PER_TILE)],
                    sx_hbm.at[pl.ds(out0, ROWS_PER_TILE)])
    pltpu.sync_copy(acc_e.at[pl.ds(r0, ROWS_PER_TILE)],
                    se_hbm.at[pl.ds(out0, ROWS_PER_TILE)])
    pltpu.sync_copy(acc_d.at[pl.ds(r0, ROWS_PER_TILE)],
                    sd_hbm.at[pl.ds(out0, ROWS_PER_TILE)])

  return k(x, packed2d, ea3d, zx, ze, zd, ones)


def _tc_body(x_r, sx_r, se_r, sd_r, w1_r, w2_r, ww1_r, ww2_r, wb_r, out_r):
  sx = sx_r[0] + sx_r[1]
  se = se_r[0] + se_r[1]
  sd = sd_r[0] + sd_r[1]
  invd = 1.0 / jnp.maximum(sd[:, 0:1], 1.0)
  hn = (jnp.dot(sx * invd, w1_r[...], preferred_element_type=jnp.float32)
        + jnp.dot(se * invd, w2_r[...], preferred_element_type=jnp.float32))
  out = (jnp.dot(x_r[...], ww1_r[...], preferred_element_type=jnp.float32)
         + jnp.dot(hn, ww2_r[...], preferred_element_type=jnp.float32)
         + wb_r[...])
  out_r[...] = out


def _tc_combine(x, sums_x, sums_e, sums_d, w1, w2, ww1, ww2, wb):
  blk = 1000
  grid = N_NODES // blk
  return pl.pallas_call(
      _tc_body,
      grid=(grid,),
      in_specs=[
          pl.BlockSpec((blk, D_FEAT), lambda i: (i, 0)),
          pl.BlockSpec((NC, blk, D_FEAT), lambda i: (0, i, 0)),
          pl.BlockSpec((NC, blk, D_EDGE), lambda i: (0, i, 0)),
          pl.BlockSpec((NC, blk, D_DEG), lambda i: (0, i, 0)),
          pl.BlockSpec((D_FEAT, D_OUT), lambda i: (0, 0)),
          pl.BlockSpec((D_EDGE, D_OUT), lambda i: (0, 0)),
          pl.BlockSpec((D_FEAT, D_OUT), lambda i: (0, 0)),
          pl.BlockSpec((D_OUT, D_OUT), lambda i: (0, 0)),
          pl.BlockSpec((1, D_OUT), lambda i: (0, 0)),
      ],
      out_specs=pl.BlockSpec((blk, D_OUT), lambda i: (i, 0)),
      out_shape=jax.ShapeDtypeStruct((N_NODES, D_OUT), jnp.float32),
  )(x, sums_x, sums_e, sums_d, w1, w2, ww1, ww2, wb)


def kernel(x, edge_index, edge_attr, weight, W_w, W_b, bias):
  src = edge_index[0].astype(jnp.int32)
  dst = edge_index[1].astype(jnp.int32)
  pad = E_PAD - N_EDGES
  src_p = jnp.concatenate([src, jnp.zeros((pad,), jnp.int32)])
  dst_p = jnp.concatenate([dst, jnp.full((pad,), N_NODES, jnp.int32)])
  packed = jnp.bitwise_or(jnp.left_shift(dst_p, 16), src_p)
  packed2d = packed.reshape(TOT_CHUNKS * C // 128, 128)
  ea128 = edge_attr.reshape(N_EDGES * D_EDGE // 128, 128)
  zx = jnp.zeros((ROWS_PER_TILE // 2, D_FEAT), jnp.float32)
  ze = jnp.zeros((ROWS_PER_TILE, D_EDGE), jnp.float32)
  zd = jnp.zeros((ROWS_PER_TILE, D_DEG), jnp.float32)
  ones = jnp.ones((C, D_DEG), jnp.float32)

  sx, se, sd = _sc_segment_sums(x, packed2d, ea128, zx, ze, zd, ones)
  sums_x = sx.reshape(NC, N_PAD, D_FEAT)
  sums_e = se.reshape(NC, N_PAD, D_EDGE)
  sums_d = sd.reshape(NC, N_PAD, D_DEG)

  wb = (W_b + bias).reshape(1, D_OUT)
  return _tc_combine(x, sums_x, sums_e, sums_d,
                     weight[:D_FEAT], weight[D_FEAT:],
                     W_w[:D_FEAT], W_w[D_FEAT:], wb)


# speedup vs baseline: 1.1902x; 1.1902x over previous
import functools

import jax
import jax.numpy as jnp
from jax import lax
from jax.experimental import pallas as pl
from jax.experimental.pallas import tpu as pltpu
from jax.experimental.pallas import tpu_sc as plsc

N_NODES = 10000
N_EDGES = 320000
D_FEAT = 128
D_EDGE = 16
D_OUT = 128

NC = 2
NS = 16
NW = NC * NS
C = 64
CH0 = 232
CH1 = 88
TOT_CHUNKS = NS * (CH0 + CH1)
E_PAD = TOT_CHUNKS * C
PK_HALF = 58
N_PAD = 10112
ROWS_PER_TILE = N_PAD // NS
D_DEG = 8
L = 16


def _sc_segment_sums(x, packed2d, ea3d, zx, ze, zd, ones):
  mesh = plsc.VectorSubcoreMesh(core_axis_name="c", subcore_axis_name="s")

  @functools.partial(
      pl.kernel,
      mesh=mesh,
      compiler_params=pltpu.CompilerParams(use_tc_tiling_on_sc=False),
      out_type=[
          jax.ShapeDtypeStruct((NC * N_PAD, D_FEAT), jnp.float32),
          jax.ShapeDtypeStruct((NC * N_PAD, D_EDGE), jnp.float32),
          jax.ShapeDtypeStruct((NC * N_PAD, D_DEG), jnp.float32),
      ],
      scratch_types=[
          pltpu.VMEM_SHARED((N_PAD, D_FEAT), jnp.float32),
          pltpu.VMEM_SHARED((N_PAD, D_EDGE), jnp.float32),
          pltpu.VMEM_SHARED((N_PAD, D_DEG), jnp.float32),
          pltpu.VMEM((PK_HALF, 128), jnp.int32),
          pltpu.VMEM((4, C), jnp.int32),
          pltpu.VMEM((4, C), jnp.int32),
          pltpu.VMEM((C, D_FEAT), jnp.float32),
          pltpu.VMEM((C, D_FEAT), jnp.float32),
          pltpu.VMEM((8, 128), jnp.float32),
          pltpu.VMEM((8, 128), jnp.float32),
          pltpu.VMEM((C, D_EDGE), jnp.float32),
          pltpu.VMEM((C, D_EDGE), jnp.float32),
          pltpu.VMEM((C, D_DEG), jnp.float32),
          pltpu.SemaphoreType.DMA,
          pltpu.SemaphoreType.DMA,
          pltpu.SemaphoreType.DMA,
          pltpu.SemaphoreType.DMA,
          pltpu.SemaphoreType.DMA,
          pltpu.SemaphoreType.DMA,
          pltpu.SemaphoreType.DMA,
          pltpu.SemaphoreType.DMA,
          pltpu.SemaphoreType.DMA,
          pltpu.SemaphoreType.DMA,
      ],
  )
  def k(x_hbm, pk_hbm, ea_hbm, zx_hbm, ze_hbm, zd_hbm, ones_hbm,
        sx_hbm, se_hbm, sd_hbm,
        acc_x, acc_e, acc_d, pk_v, src_i, dst_i,
        rows0, rows1, eaw0, eaw1, ea0, ea1, ones_v,
        sg0, sg1, se0, se1, ssx0, ssx1, sse0, sse1, ssd0, ssd1):
    cid = lax.axis_index("c")
    sid = lax.axis_index("s")
    nch = jnp.where(cid == 0, CH0, CH1)
    cbase = jnp.where(cid == 0, sid * CH0, NS * CH0 + sid * CH1)
    rows = (rows0, rows1)
    eaw = (eaw0, eaw1)
    eab = (ea0, ea1)
    sg = (sg0, sg1)
    sea = (se0, se1)
    ssx = (ssx0, ssx1)
    sse = (sse0, sse1)
    ssd = (ssd0, ssd1)

    r0 = sid * ROWS_PER_TILE
    half = ROWS_PER_TILE // 2
    for t in range(2):
      pltpu.sync_copy(zx_hbm, acc_x.at[pl.ds(r0 + t * half, half)])
    pltpu.sync_copy(ze_hbm, acc_e.at[pl.ds(r0, ROWS_PER_TILE)])
    pltpu.sync_copy(zd_hbm, acc_d.at[pl.ds(r0, ROWS_PER_TILE)])

    @pl.when(cid == 0)
    def _():
      pltpu.sync_copy(pk_hbm.at[pl.ds(cbase // 2, PK_HALF)],
                      pk_v.at[pl.ds(0, PK_HALF)])

    @pl.when(cid == 1)
    def _():
      pltpu.sync_copy(pk_hbm.at[pl.ds(cbase // 2, CH1 // 2)],
                      pk_v.at[pl.ds(0, CH1 // 2)])
    pltpu.sync_copy(ones_hbm, ones_v)
    plsc.subcore_barrier()

    def unpack(row, col0, slot):
      for kk in range(C // L):
        pk = pk_v[row, pl.ds(col0 + kk * L, L)]
        src_i[slot, pl.ds(kk * L, L)] = pk & 0xFFFF
        dst_i[slot, pl.ds(kk * L, L)] = lax.shift_right_logical(pk, 16)

    def ea_row(n):
      g = cbase + n
      return jnp.where(g < N_EDGES // C, g * 8, 0)

    def issue_gather(n, p, slot):
      pltpu.async_copy(x_hbm.at[src_i.at[slot]], rows[p], sg[p])
      pltpu.async_copy(ea_hbm.at[pl.ds(ea_row(n), 8)], eaw[p], sea[p])

    def wait_and_scatter(n, p, slot):
      pltpu.make_async_copy(x_hbm.at[src_i.at[slot]], rows[p], sg[p]).wait()
      pltpu.make_async_copy(ea_hbm.at[pl.ds(ea_row(n), 8)], eaw[p],
                            sea[p]).wait()
      for r in range(C):
        eab[p][r, pl.ds(0, L)] = eaw[p][r // 8, pl.ds((r % 8) * L, L)]
      pltpu.async_copy(rows[p], acc_x.at[dst_i.at[slot]], ssx[p], add=True)
      pltpu.async_copy(eab[p], acc_e.at[dst_i.at[slot]], sse[p], add=True)
      pltpu.async_copy(ones_v, acc_d.at[dst_i.at[slot]], ssd[p], add=True)

    def drain_scatter(p, slot):
      pltpu.make_async_copy(rows[p], acc_x.at[dst_i.at[slot]], ssx[p]).wait()
      pltpu.make_async_copy(eab[p], acc_e.at[dst_i.at[slot]], sse[p]).wait()
      pltpu.make_async_copy(ones_v, acc_d.at[dst_i.at[slot]], ssd[p]).wait()

    for q in range(4):
      unpack(q // 2, 64 * (q % 2), q)
    issue_gather(0, 0, 0)
    issue_gather(1, 1, 1)

    @pl.loop(0, nch, step=4)
    def _body(j):
      @pl.when(j + 4 == 2 * PK_HALF)
      def _():
        pltpu.sync_copy(pk_hbm.at[pl.ds(cbase // 2 + PK_HALF, PK_HALF)],
                        pk_v.at[pl.ds(0, PK_HALF)])

      jrow = j // 2 - jnp.where(j + 4 >= 2 * PK_HALF, PK_HALF, 0)
      wait_and_scatter(j, 0, 0)
      wait_and_scatter(j + 1, 1, 1)
      drain_scatter(0, 0)

      @pl.when(j + 4 < nch)
      def _():
        unpack(jrow + 2, 0, 0)
      issue_gather(j + 2, 0, 2)

      drain_scatter(1, 1)

      @pl.when(j + 4 < nch)
      def _():
        unpack(jrow + 2, 64, 1)
      issue_gather(j + 3, 1, 3)

      wait_and_scatter(j + 2, 0, 2)
      wait_and_scatter(j + 3, 1, 3)

      drain_scatter(0, 2)

      @pl.when(j + 4 < nch)
      def _():
        unpack(jrow + 3, 0, 2)
        issue_gather(j + 4, 0, 0)

      drain_scatter(1, 3)

      @pl.when(j + 4 < nch)
      def _():
        unpack(jrow + 3, 64, 3)
        issue_gather(j + 5, 1, 1)

    plsc.subcore_barrier()

    out0 = cid * N_PAD + r0
    pltpu.sync_copy(acc_x.at[pl.ds(r0, ROWS_PER_TILE)],
                    sx_hbm.at[pl.ds(out0, ROWS_PER_TILE)])
    pltpu.sync_copy(acc_e.at[pl.ds(r0, ROWS_PER_TILE)],
                    se_hbm.at[pl.ds(out0, ROWS_PER_TILE)])
    pltpu.sync_copy(acc_d.at[pl.ds(r0, ROWS_PER_TILE)],
                    sd_hbm.at[pl.ds(out0, ROWS_PER_TILE)])

  return k(x, packed2d, ea3d, zx, ze, zd, ones)


def _tc_body(x_r, sx_r, se_r, sd_r, w1_r, w2_r, ww1_r, ww2_r, wb_r, out_r):
  sx = sx_r[0] + sx_r[1]
  se = se_r[0] + se_r[1]
  sd = sd_r[0] + sd_r[1]
  invd = 1.0 / jnp.maximum(sd[:, 0:1], 1.0)
  hn = (jnp.dot(sx * invd, w1_r[...], preferred_element_type=jnp.float32)
        + jnp.dot(se * invd, w2_r[...], preferred_element_type=jnp.float32))
  out = (jnp.dot(x_r[...], ww1_r[...], preferred_element_type=jnp.float32)
         + jnp.dot(hn, ww2_r[...], preferred_element_type=jnp.float32)
         + wb_r[...])
  out_r[...] = out


def _tc_combine(x, sums_x, sums_e, sums_d, w1, w2, ww1, ww2, wb):
  blk = 1000
  grid = N_NODES // blk
  return pl.pallas_call(
      _tc_body,
      grid=(grid,),
      in_specs=[
          pl.BlockSpec((blk, D_FEAT), lambda i: (i, 0)),
          pl.BlockSpec((NC, blk, D_FEAT), lambda i: (0, i, 0)),
          pl.BlockSpec((NC, blk, D_EDGE), lambda i: (0, i, 0)),
          pl.BlockSpec((NC, blk, D_DEG), lambda i: (0, i, 0)),
          pl.BlockSpec((D_FEAT, D_OUT), lambda i: (0, 0)),
          pl.BlockSpec((D_EDGE, D_OUT), lambda i: (0, 0)),
          pl.BlockSpec((D_FEAT, D_OUT), lambda i: (0, 0)),
          pl.BlockSpec((D_OUT, D_OUT), lambda i: (0, 0)),
          pl.BlockSpec((1, D_OUT), lambda i: (0, 0)),
      ],
      out_specs=pl.BlockSpec((blk, D_OUT), lambda i: (i, 0)),
      out_shape=jax.ShapeDtypeStruct((N_NODES, D_OUT), jnp.float32),
  )(x, sums_x, sums_e, sums_d, w1, w2, ww1, ww2, wb)


def kernel(x, edge_index, edge_attr, weight, W_w, W_b, bias):
  src = edge_index[0].astype(jnp.int32)
  dst = edge_index[1].astype(jnp.int32)
  pad = E_PAD - N_EDGES
  src_p = jnp.concatenate([src, jnp.zeros((pad,), jnp.int32)])
  dst_p = jnp.concatenate([dst, jnp.full((pad,), N_NODES, jnp.int32)])
  packed = jnp.bitwise_or(jnp.left_shift(dst_p, 16), src_p)
  packed2d = packed.reshape(TOT_CHUNKS * C // 128, 128)
  ea128 = edge_attr.reshape(N_EDGES * D_EDGE // 128, 128)
  zx = jnp.zeros((ROWS_PER_TILE // 2, D_FEAT), jnp.float32)
  ze = jnp.zeros((ROWS_PER_TILE, D_EDGE), jnp.float32)
  zd = jnp.zeros((ROWS_PER_TILE, D_DEG), jnp.float32)
  ones = jnp.ones((C, D_DEG), jnp.float32)

  sx, se, sd = _sc_segment_sums(x, packed2d, ea128, zx, ze, zd, ones)
  sums_x = sx.reshape(NC, N_PAD, D_FEAT)
  sums_e = se.reshape(NC, N_PAD, D_EDGE)
  sums_d = sd.reshape(NC, N_PAD, D_DEG)

  wb = (W_b + bias).reshape(1, D_OUT)
  return _tc_combine(x, sums_x, sums_e, sums_d,
                     weight[:D_FEAT], weight[D_FEAT:],
                     W_w[:D_FEAT], W_w[D_FEAT:], wb)


# speedup vs baseline: 2.0286x; 1.7044x over previous
import functools

import jax
import jax.numpy as jnp
from jax import lax
from jax.experimental import pallas as pl
from jax.experimental.pallas import tpu as pltpu
from jax.experimental.pallas import tpu_sc as plsc

N_NODES = 10000
N_EDGES = 320000
D_FEAT = 128
D_EDGE = 16
D_OUT = 128

NC = 2
NS = 16
NW = NC * NS
C = 64
CH0 = 160
CH1 = 160
TOT_CHUNKS = NS * (CH0 + CH1)
E_PAD = TOT_CHUNKS * C
PK_HALF = 40
N_PAD = 10112
ROWS_PER_TILE = N_PAD // NS
D_DEG = 8
L = 16


def _sc_segment_sums(x, packed2d, ea3d, zx, ze, zd, ones):
  mesh = plsc.VectorSubcoreMesh(core_axis_name="c", subcore_axis_name="s")

  @functools.partial(
      pl.kernel,
      mesh=mesh,
      compiler_params=pltpu.CompilerParams(use_tc_tiling_on_sc=False),
      out_type=[
          jax.ShapeDtypeStruct((NC * N_PAD, D_FEAT), jnp.float32),
          jax.ShapeDtypeStruct((NC * N_PAD, D_EDGE), jnp.float32),
          jax.ShapeDtypeStruct((NC * N_PAD, D_DEG), jnp.float32),
      ],
      scratch_types=[
          pltpu.VMEM_SHARED((N_PAD, D_FEAT), jnp.float32),
          pltpu.VMEM_SHARED((N_PAD, D_EDGE), jnp.float32),
          pltpu.VMEM_SHARED((N_PAD, D_DEG), jnp.float32),
          pltpu.VMEM((PK_HALF, 128), jnp.int32),
          pltpu.VMEM((4, C), jnp.int32),
          pltpu.VMEM((4, C), jnp.int32),
          pltpu.VMEM((C, D_FEAT), jnp.float32),
          pltpu.VMEM((C, D_FEAT), jnp.float32),
          pltpu.VMEM((8, 128), jnp.float32),
          pltpu.VMEM((8, 128), jnp.float32),
          pltpu.VMEM((C, D_EDGE), jnp.float32),
          pltpu.VMEM((C, D_EDGE), jnp.float32),
          pltpu.VMEM((C, D_DEG), jnp.float32),
          pltpu.SemaphoreType.DMA,
          pltpu.SemaphoreType.DMA,
          pltpu.SemaphoreType.DMA,
          pltpu.SemaphoreType.DMA,
          pltpu.SemaphoreType.DMA,
          pltpu.SemaphoreType.DMA,
          pltpu.SemaphoreType.DMA,
          pltpu.SemaphoreType.DMA,
          pltpu.SemaphoreType.DMA,
          pltpu.SemaphoreType.DMA,
      ],
  )
  def k(x_hbm, pk_hbm, ea_hbm, zx_hbm, ze_hbm, zd_hbm, ones_hbm,
        sx_hbm, se_hbm, sd_hbm,
        acc_x, acc_e, acc_d, pk_v, src_i, dst_i,
        rows0, rows1, eaw0, eaw1, ea0, ea1, ones_v,
        sg0, sg1, se0, se1, ssx0, ssx1, sse0, sse1, ssd0, ssd1):
    cid = lax.axis_index("c")
    sid = lax.axis_index("s")
    nch = jnp.where(cid == 0, CH0, CH1)
    cbase = jnp.where(cid == 0, sid * CH0, NS * CH0 + sid * CH1)
    rows = (rows0, rows1)
    eaw = (eaw0, eaw1)
    eab = (ea0, ea1)
    sg = (sg0, sg1)
    sea = (se0, se1)
    ssx = (ssx0, ssx1)
    sse = (sse0, sse1)
    ssd = (ssd0, ssd1)

    r0 = sid * ROWS_PER_TILE
    half = ROWS_PER_TILE // 2
    for t in range(2):
      pltpu.sync_copy(zx_hbm, acc_x.at[pl.ds(r0 + t * half, half)])
    pltpu.sync_copy(ze_hbm, acc_e.at[pl.ds(r0, ROWS_PER_TILE)])
    pltpu.sync_copy(zd_hbm, acc_d.at[pl.ds(r0, ROWS_PER_TILE)])

    pltpu.sync_copy(pk_hbm.at[pl.ds(cbase // 2, PK_HALF)],
                    pk_v.at[pl.ds(0, PK_HALF)])
    pltpu.sync_copy(ones_hbm, ones_v)
    plsc.subcore_barrier()

    def unpack(row, col0, slot):
      for kk in range(C // L):
        pk = pk_v[row, pl.ds(col0 + kk * L, L)]
        src_i[slot, pl.ds(kk * L, L)] = pk & 0xFFFF
        dst_i[slot, pl.ds(kk * L, L)] = lax.shift_right_logical(pk, 16)

    def ea_row(n):
      g = cbase + n
      return jnp.where(g < N_EDGES // C, g * 8, 0)

    def issue_gather(n, p, slot):
      pltpu.async_copy(x_hbm.at[src_i.at[slot]], rows[p], sg[p])
      pltpu.async_copy(ea_hbm.at[pl.ds(ea_row(n), 8)], eaw[p], sea[p])

    def wait_and_scatter(n, p, slot):
      pltpu.make_async_copy(x_hbm.at[src_i.at[slot]], rows[p], sg[p]).wait()
      pltpu.make_async_copy(ea_hbm.at[pl.ds(ea_row(n), 8)], eaw[p],
                            sea[p]).wait()
      for r in range(C):
        eab[p][r, pl.ds(0, L)] = eaw[p][r // 8, pl.ds((r % 8) * L, L)]
      pltpu.async_copy(rows[p], acc_x.at[dst_i.at[slot]], ssx[p], add=True)
      pltpu.async_copy(eab[p], acc_e.at[dst_i.at[slot]], sse[p], add=True)
      pltpu.async_copy(ones_v, acc_d.at[dst_i.at[slot]], ssd[p], add=True)

    def drain_scatter(p, slot):
      pltpu.make_async_copy(rows[p], acc_x.at[dst_i.at[slot]], ssx[p]).wait()
      pltpu.make_async_copy(eab[p], acc_e.at[dst_i.at[slot]], sse[p]).wait()
      pltpu.make_async_copy(ones_v, acc_d.at[dst_i.at[slot]], ssd[p]).wait()

    for q in range(4):
      unpack(q // 2, 64 * (q % 2), q)
    issue_gather(0, 0, 0)
    issue_gather(1, 1, 1)

    @pl.loop(0, nch, step=4)
    def _body(j):
      @pl.when(j + 4 == 2 * PK_HALF)
      def _():
        pltpu.sync_copy(pk_hbm.at[pl.ds(cbase // 2 + PK_HALF, PK_HALF)],
                        pk_v.at[pl.ds(0, PK_HALF)])

      jrow = j // 2 - jnp.where(j + 4 >= 2 * PK_HALF, PK_HALF, 0)
      wait_and_scatter(j, 0, 0)
      wait_and_scatter(j + 1, 1, 1)
      drain_scatter(0, 0)

      @pl.when(j + 4 < nch)
      def _():
        unpack(jrow + 2, 0, 0)
      issue_gather(j + 2, 0, 2)

      drain_scatter(1, 1)

      @pl.when(j + 4 < nch)
      def _():
        unpack(jrow + 2, 64, 1)
      issue_gather(j + 3, 1, 3)

      wait_and_scatter(j + 2, 0, 2)
      wait_and_scatter(j + 3, 1, 3)

      drain_scatter(0, 2)

      @pl.when(j + 4 < nch)
      def _():
        unpack(jrow + 3, 0, 2)
        issue_gather(j + 4, 0, 0)

      drain_scatter(1, 3)

      @pl.when(j + 4 < nch)
      def _():
        unpack(jrow + 3, 64, 3)
        issue_gather(j + 5, 1, 1)

    plsc.subcore_barrier()

    out0 = cid * N_PAD + r0
    pltpu.sync_copy(acc_x.at[pl.ds(r0, ROWS_PER_TILE)],
                    sx_hbm.at[pl.ds(out0, ROWS_PER_TILE)])
    pltpu.sync_copy(acc_e.at[pl.ds(r0, ROWS_PER_TILE)],
                    se_hbm.at[pl.ds(out0, ROWS_PER_TILE)])
    pltpu.sync_copy(acc_d.at[pl.ds(r0, ROWS_PER_TILE)],
                    sd_hbm.at[pl.ds(out0, ROWS_PER_TILE)])

  return k(x, packed2d, ea3d, zx, ze, zd, ones)


def _tc_body(x_r, sx_r, se_r, sd_r, w1_r, w2_r, ww1_r, ww2_r, wb_r, out_r):
  sx = sx_r[0] + sx_r[1]
  se = se_r[0] + se_r[1]
  sd = sd_r[0] + sd_r[1]
  invd = 1.0 / jnp.maximum(sd[:, 0:1], 1.0)
  hn = (jnp.dot(sx * invd, w1_r[...], preferred_element_type=jnp.float32)
        + jnp.dot(se * invd, w2_r[...], preferred_element_type=jnp.float32))
  out = (jnp.dot(x_r[...], ww1_r[...], preferred_element_type=jnp.float32)
         + jnp.dot(hn, ww2_r[...], preferred_element_type=jnp.float32)
         + wb_r[...])
  out_r[...] = out


def _tc_combine(x, sums_x, sums_e, sums_d, w1, w2, ww1, ww2, wb):
  blk = 1000
  grid = N_NODES // blk
  return pl.pallas_call(
      _tc_body,
      grid=(grid,),
      in_specs=[
          pl.BlockSpec((blk, D_FEAT), lambda i: (i, 0)),
          pl.BlockSpec((NC, blk, D_FEAT), lambda i: (0, i, 0)),
          pl.BlockSpec((NC, blk, D_EDGE), lambda i: (0, i, 0)),
          pl.BlockSpec((NC, blk, D_DEG), lambda i: (0, i, 0)),
          pl.BlockSpec((D_FEAT, D_OUT), lambda i: (0, 0)),
          pl.BlockSpec((D_EDGE, D_OUT), lambda i: (0, 0)),
          pl.BlockSpec((D_FEAT, D_OUT), lambda i: (0, 0)),
          pl.BlockSpec((D_OUT, D_OUT), lambda i: (0, 0)),
          pl.BlockSpec((1, D_OUT), lambda i: (0, 0)),
      ],
      out_specs=pl.BlockSpec((blk, D_OUT), lambda i: (i, 0)),
      out_shape=jax.ShapeDtypeStruct((N_NODES, D_OUT), jnp.float32),
  )(x, sums_x, sums_e, sums_d, w1, w2, ww1, ww2, wb)


def kernel(x, edge_index, edge_attr, weight, W_w, W_b, bias):
  src = edge_index[0].astype(jnp.int32)
  dst = edge_index[1].astype(jnp.int32)
  pad = E_PAD - N_EDGES
  fill = jnp.arange(pad, dtype=jnp.int32)
  src_p = jnp.concatenate([src, fill % N_NODES])
  dst_p = jnp.concatenate([dst, N_NODES + fill % (N_PAD - N_NODES)])
  packed = jnp.bitwise_or(jnp.left_shift(dst_p, 16), src_p)
  packed2d = packed.reshape(TOT_CHUNKS * C // 128, 128)
  ea128 = edge_attr.reshape(N_EDGES * D_EDGE // 128, 128)
  zx = jnp.zeros((ROWS_PER_TILE // 2, D_FEAT), jnp.float32)
  ze = jnp.zeros((ROWS_PER_TILE, D_EDGE), jnp.float32)
  zd = jnp.zeros((ROWS_PER_TILE, D_DEG), jnp.float32)
  ones = jnp.ones((C, D_DEG), jnp.float32)

  sx, se, sd = _sc_segment_sums(x, packed2d, ea128, zx, ze, zd, ones)
  sums_x = sx.reshape(NC, N_PAD, D_FEAT)
  sums_e = se.reshape(NC, N_PAD, D_EDGE)
  sums_d = sd.reshape(NC, N_PAD, D_DEG)

  wb = (W_b + bias).reshape(1, D_OUT)
  return _tc_combine(x, sums_x, sums_e, sums_d,
                     weight[:D_FEAT], weight[D_FEAT:],
                     W_w[:D_FEAT], W_w[D_FEAT:], wb)


# speedup vs baseline: 2.0905x; 1.0305x over previous
import functools

import jax
import jax.numpy as jnp
from jax import lax
from jax.experimental import pallas as pl
from jax.experimental.pallas import tpu as pltpu
from jax.experimental.pallas import tpu_sc as plsc

N_NODES = 10000
N_EDGES = 320000
D_FEAT = 128
D_EDGE = 16
D_OUT = 128

NC = 2
NS = 16
NW = NC * NS
C = 128
CHUNKS = 80
TOT_CHUNKS = NW * CHUNKS
E_PAD = TOT_CHUNKS * C
PK_HALF = 40
N_PAD = 10112
ROWS_PER_TILE = N_PAD // NS
D_ED = 24
L = 16

_mesh = plsc.VectorSubcoreMesh(core_axis_name="c", subcore_axis_name="s")
_sc_params = pltpu.CompilerParams(use_tc_tiling_on_sc=False)


def _sc_gather_scatter_x(x, packed2d, zx):
  @functools.partial(
      pl.kernel,
      mesh=_mesh,
      compiler_params=_sc_params,
      out_type=jax.ShapeDtypeStruct((NC, N_PAD, D_FEAT), jnp.float32),
      scratch_types=[
          pltpu.VMEM_SHARED((N_PAD, D_FEAT), jnp.float32),
          pltpu.VMEM((PK_HALF, 128), jnp.int32),
          pltpu.VMEM((4, C), jnp.int32),
          pltpu.VMEM((4, C), jnp.int32),
          pltpu.VMEM((C, D_FEAT), jnp.float32),
          pltpu.VMEM((C, D_FEAT), jnp.float32),
          pltpu.SemaphoreType.DMA,
          pltpu.SemaphoreType.DMA,
          pltpu.SemaphoreType.DMA,
          pltpu.SemaphoreType.DMA,
      ],
  )
  def k(x_hbm, pk_hbm, zx_hbm, sx_hbm,
        acc_x, pk_v, src_i, dst_i, rows0, rows1,
        sg0, sg1, ssx0, ssx1):
    cid = lax.axis_index("c")
    sid = lax.axis_index("s")
    wid = sid * NC + cid
    cbase = wid * CHUNKS
    rows = (rows0, rows1)
    sg = (sg0, sg1)
    ssx = (ssx0, ssx1)

    r0 = sid * ROWS_PER_TILE
    half = ROWS_PER_TILE // 2
    for t in range(2):
      pltpu.sync_copy(zx_hbm, acc_x.at[pl.ds(r0 + t * half, half)])

    pltpu.sync_copy(pk_hbm.at[pl.ds(cbase, PK_HALF)],
                    pk_v.at[pl.ds(0, PK_HALF)])
    plsc.subcore_barrier()

    def unpack(row, slot):
      for kk in range(C // L):
        pk = pk_v[row, pl.ds(kk * L, L)]
        src_i[slot, pl.ds(kk * L, L)] = pk & 0xFFFF
        dst_i[slot, pl.ds(kk * L, L)] = lax.shift_right_logical(pk, 16)

    def issue_gather(p, slot):
      pltpu.async_copy(x_hbm.at[src_i.at[slot]], rows[p], sg[p])

    def wait_and_scatter(p, slot):
      pltpu.make_async_copy(x_hbm.at[src_i.at[slot]], rows[p], sg[p]).wait()
      pltpu.async_copy(rows[p], acc_x.at[dst_i.at[slot]], ssx[p], add=True)

    def drain_scatter(p, slot):
      pltpu.make_async_copy(rows[p], acc_x.at[dst_i.at[slot]], ssx[p]).wait()

    for q in range(4):
      unpack(q, q)
    issue_gather(0, 0)
    issue_gather(1, 1)

    @pl.loop(0, CHUNKS, step=4)
    def _body(j):
      @pl.when(j + 4 == PK_HALF)
      def _():
        pltpu.sync_copy(pk_hbm.at[pl.ds(cbase + PK_HALF, PK_HALF)],
                        pk_v.at[pl.ds(0, PK_HALF)])

      jrow = j - jnp.where(j + 4 >= PK_HALF, PK_HALF, 0)
      wait_and_scatter(0, 0)
      wait_and_scatter(1, 1)
      drain_scatter(0, 0)

      @pl.when(j + 4 < CHUNKS)
      def _():
        unpack(jrow + 4, 0)
      issue_gather(0, 2)

      drain_scatter(1, 1)

      @pl.when(j + 4 < CHUNKS)
      def _():
        unpack(jrow + 5, 1)
      issue_gather(1, 3)

      wait_and_scatter(0, 2)
      wait_and_scatter(1, 3)
      drain_scatter(0, 2)

      @pl.when(j + 4 < CHUNKS)
      def _():
        unpack(jrow + 6, 2)
        issue_gather(0, 0)

      drain_scatter(1, 3)

      @pl.when(j + 4 < CHUNKS)
      def _():
        unpack(jrow + 7, 3)
        issue_gather(1, 1)

    plsc.subcore_barrier()
    pltpu.sync_copy(acc_x.at[pl.ds(r0, ROWS_PER_TILE)],
                    sx_hbm.at[cid, pl.ds(r0, ROWS_PER_TILE)])

  return k(x, packed2d, zx)


def _sc_scatter_ea(ea128, packed2d, ze):
  @functools.partial(
      pl.kernel,
      mesh=_mesh,
      compiler_params=_sc_params,
      out_type=jax.ShapeDtypeStruct((NC, N_PAD, D_ED), jnp.float32),
      scratch_types=[
          pltpu.VMEM_SHARED((N_PAD, D_ED), jnp.float32),
          pltpu.VMEM((PK_HALF, 128), jnp.int32),
          pltpu.VMEM((4, C), jnp.int32),
          pltpu.VMEM((C // 8, 128), jnp.float32),
          pltpu.VMEM((C // 8, 128), jnp.float32),
          pltpu.VMEM((C, D_ED), jnp.float32),
          pltpu.VMEM((C, D_ED), jnp.float32),
          pltpu.SemaphoreType.DMA,
          pltpu.SemaphoreType.DMA,
          pltpu.SemaphoreType.DMA,
          pltpu.SemaphoreType.DMA,
      ],
  )
  def k(ea_hbm, pk_hbm, ze_hbm, se_hbm,
        acc_e, pk_v, dst_i, eaw0, eaw1, ea0, ea1,
        sl0, sl1, ss0, ss1):
    cid = lax.axis_index("c")
    sid = lax.axis_index("s")
    wid = sid * NC + cid
    cbase = wid * CHUNKS
    eaw = (eaw0, eaw1)
    eab = (ea0, ea1)
    sl = (sl0, sl1)
    ss = (ss0, ss1)

    r0 = sid * ROWS_PER_TILE
    pltpu.sync_copy(ze_hbm, acc_e.at[pl.ds(r0, ROWS_PER_TILE)])
    for rr in range(C):
      ea0[rr, pl.ds(8, L)] = jnp.ones((L,), jnp.float32)
      ea1[rr, pl.ds(8, L)] = jnp.ones((L,), jnp.float32)

    pltpu.sync_copy(pk_hbm.at[pl.ds(cbase, PK_HALF)],
                    pk_v.at[pl.ds(0, PK_HALF)])
    plsc.subcore_barrier()

    def unpack(row, slot):
      for kk in range(C // L):
        pk = pk_v[row, pl.ds(kk * L, L)]
        dst_i[slot, pl.ds(kk * L, L)] = lax.shift_right_logical(pk, 16)

    def ea_row(n):
      g = cbase + n
      return jnp.where(g < N_EDGES // C, g * (C // 8), 0)

    def issue_load(n, p):
      pltpu.async_copy(ea_hbm.at[pl.ds(ea_row(n), C // 8)], eaw[p], sl[p])

    def wait_and_scatter(n, p, slot):
      pltpu.make_async_copy(ea_hbm.at[pl.ds(ea_row(n), C // 8)], eaw[p],
                            sl[p]).wait()
      for r in range(C):
        eab[p][r, pl.ds(0, L)] = eaw[p][r // 8, pl.ds((r % 8) * L, L)]
      pltpu.async_copy(eab[p], acc_e.at[dst_i.at[slot]], ss[p], add=True)

    def drain_scatter(p, slot):
      pltpu.make_async_copy(eab[p], acc_e.at[dst_i.at[slot]], ss[p]).wait()

    for q in range(4):
      unpack(q, q)
    issue_load(0, 0)
    issue_load(1, 1)

    @pl.loop(0, CHUNKS, step=4)
    def _body(j):
      @pl.when(j + 4 == PK_HALF)
      def _():
        pltpu.sync_copy(pk_hbm.at[pl.ds(cbase + PK_HALF, PK_HALF)],
                        pk_v.at[pl.ds(0, PK_HALF)])

      jrow = j - jnp.where(j + 4 >= PK_HALF, PK_HALF, 0)
      wait_and_scatter(j, 0, 0)
      issue_load(j + 2, 0)
      wait_and_scatter(j + 1, 1, 1)
      issue_load(j + 3, 1)
      drain_scatter(0, 0)

      @pl.when(j + 4 < CHUNKS)
      def _():
        unpack(jrow + 4, 0)

      drain_scatter(1, 1)

      @pl.when(j + 4 < CHUNKS)
      def _():
        unpack(jrow + 5, 1)

      wait_and_scatter(j + 2, 0, 2)

      @pl.when(j + 4 < CHUNKS)
      def _():
        issue_load(j + 4, 0)
      wait_and_scatter(j + 3, 1, 3)

      @pl.when(j + 4 < CHUNKS)
      def _():
        issue_load(j + 5, 1)
      drain_scatter(0, 2)

      @pl.when(j + 4 < CHUNKS)
      def _():
        unpack(jrow + 6, 2)

      drain_scatter(1, 3)

      @pl.when(j + 4 < CHUNKS)
      def _():
        unpack(jrow + 7, 3)

    plsc.subcore_barrier()
    pltpu.sync_copy(acc_e.at[pl.ds(r0, ROWS_PER_TILE)],
                    se_hbm.at[cid, pl.ds(r0, ROWS_PER_TILE)])

  return k(ea128, packed2d, ze)


def _tc_body(x_r, sx_r, se_r, w1_r, w2_r, ww1_r, ww2_r, wb_r, out_r):
  sx = sx_r[0] + sx_r[1]
  sed = se_r[0] + se_r[1]
  se = sed[:, 0:D_EDGE]
  invd = 1.0 / jnp.maximum(sed[:, D_EDGE:D_EDGE + 1], 1.0)
  hn = (jnp.dot(sx * invd, w1_r[...], preferred_element_type=jnp.float32)
        + jnp.dot(se * invd, w2_r[...], preferred_element_type=jnp.float32))
  out = (jnp.dot(x_r[...], ww1_r[...], preferred_element_type=jnp.float32)
         + jnp.dot(hn, ww2_r[...], preferred_element_type=jnp.float32)
         + wb_r[...])
  out_r[...] = out


def _tc_combine(x, sums_x, sums_e, w1, w2, ww1, ww2, wb):
  blk = 1000
  grid = N_NODES // blk
  return pl.pallas_call(
      _tc_body,
      grid=(grid,),
      in_specs=[
          pl.BlockSpec((blk, D_FEAT), lambda i: (i, 0)),
          pl.BlockSpec((NC, blk, D_FEAT), lambda i: (0, i, 0)),
          pl.BlockSpec((NC, blk, D_ED), lambda i: (0, i, 0)),
          pl.BlockSpec((D_FEAT, D_OUT), lambda i: (0, 0)),
          pl.BlockSpec((D_EDGE, D_OUT), lambda i: (0, 0)),
          pl.BlockSpec((D_FEAT, D_OUT), lambda i: (0, 0)),
          pl.BlockSpec((D_OUT, D_OUT), lambda i: (0, 0)),
          pl.BlockSpec((1, D_OUT), lambda i: (0, 0)),
      ],
      out_specs=pl.BlockSpec((blk, D_OUT), lambda i: (i, 0)),
      out_shape=jax.ShapeDtypeStruct((N_NODES, D_OUT), jnp.float32),
  )(x, sums_x, sums_e, w1, w2, ww1, ww2, wb)


def kernel(x, edge_index, edge_attr, weight, W_w, W_b, bias):
  src = edge_index[0].astype(jnp.int32)
  dst = edge_index[1].astype(jnp.int32)
  pad = E_PAD - N_EDGES
  fill = jnp.arange(pad, dtype=jnp.int32)
  src_p = jnp.concatenate([src, fill % N_NODES])
  dst_p = jnp.concatenate([dst, N_NODES + fill % (N_PAD - N_NODES)])
  packed = jnp.bitwise_or(jnp.left_shift(dst_p, 16), src_p)
  packed2d = packed.reshape(E_PAD // 128, 128)
  ea128 = edge_attr.reshape(N_EDGES * D_EDGE // 128, 128)
  zx = jnp.zeros((ROWS_PER_TILE // 2, D_FEAT), jnp.float32)
  ze = jnp.zeros((ROWS_PER_TILE, D_ED), jnp.float32)

  sums_x = _sc_gather_scatter_x(x, packed2d, zx)
  sums_e = _sc_scatter_ea(ea128, packed2d, ze)

  wb = (W_b + bias).reshape(1, D_OUT)
  return _tc_combine(x, sums_x, sums_e,
                     weight[:D_FEAT], weight[D_FEAT:],
                     W_w[:D_FEAT], W_w[D_FEAT:], wb)


# speedup vs baseline: 2.7078x; 1.2953x over previous
import functools

import jax
import jax.numpy as jnp
from jax import lax
from jax.experimental import pallas as pl
from jax.experimental.pallas import tpu as pltpu
from jax.experimental.pallas import tpu_sc as plsc

N_NODES = 10000
N_EDGES = 320000
D_FEAT = 128
D_EDGE = 16
D_OUT = 128

NC = 2
NS = 16
NW = NC * NS
C = 128
CHUNKS = 80
TOT_CHUNKS = NW * CHUNKS
E_PAD = TOT_CHUNKS * C
PK_HALF = 40
N_PAD = 10112
ROWS_PER_TILE = N_PAD // NS
D_ED = 24
L = 16

_mesh = plsc.VectorSubcoreMesh(core_axis_name="c", subcore_axis_name="s")
_sc_params = pltpu.CompilerParams(use_tc_tiling_on_sc=False)


def _sc_gather_scatter_x(x, packed2d, zx):
  @functools.partial(
      pl.kernel,
      mesh=_mesh,
      compiler_params=_sc_params,
      out_type=jax.ShapeDtypeStruct((NC, N_PAD, D_FEAT), jnp.float32),
      scratch_types=[
          pltpu.VMEM_SHARED((N_PAD, D_FEAT), jnp.float32),
          pltpu.VMEM((PK_HALF, 128), jnp.int32),
          pltpu.VMEM((4, C), jnp.int32),
          pltpu.VMEM((4, C), jnp.int32),
          pltpu.VMEM((C, D_FEAT), jnp.float32),
          pltpu.VMEM((C, D_FEAT), jnp.float32),
          pltpu.SemaphoreType.DMA,
          pltpu.SemaphoreType.DMA,
          pltpu.SemaphoreType.DMA,
          pltpu.SemaphoreType.DMA,
      ],
  )
  def k(x_hbm, pk_hbm, zx_hbm, sx_hbm,
        acc_x, pk_v, src_i, dst_i, rows0, rows1,
        sg0, sg1, ssx0, ssx1):
    cid = lax.axis_index("c")
    sid = lax.axis_index("s")
    wid = sid * NC + cid
    cbase = wid * CHUNKS
    rows = (rows0, rows1)
    sg = (sg0, sg1)
    ssx = (ssx0, ssx1)

    r0 = sid * ROWS_PER_TILE
    half = ROWS_PER_TILE // 2
    for t in range(2):
      pltpu.sync_copy(zx_hbm, acc_x.at[pl.ds(r0 + t * half, half)])

    pltpu.sync_copy(pk_hbm.at[pl.ds(cbase, PK_HALF)],
                    pk_v.at[pl.ds(0, PK_HALF)])
    plsc.subcore_barrier()

    def unpack(row, slot):
      for kk in range(C // L):
        pk = pk_v[row, pl.ds(kk * L, L)]
        src_i[slot, pl.ds(kk * L, L)] = pk & 0xFFFF
        dst_i[slot, pl.ds(kk * L, L)] = lax.shift_right_logical(pk, 16)

    def issue_gather(p, slot):
      pltpu.async_copy(x_hbm.at[src_i.at[slot]], rows[p], sg[p])

    def wait_and_scatter(p, slot):
      pltpu.make_async_copy(x_hbm.at[src_i.at[slot]], rows[p], sg[p]).wait()
      pltpu.async_copy(rows[p], acc_x.at[dst_i.at[slot]], ssx[p], add=True)

    def drain_scatter(p, slot):
      pltpu.make_async_copy(rows[p], acc_x.at[dst_i.at[slot]], ssx[p]).wait()

    for q in range(4):
      unpack(q, q)
    issue_gather(0, 0)
    issue_gather(1, 1)

    @pl.loop(0, CHUNKS, step=4)
    def _body(j):
      @pl.when(j + 4 == PK_HALF)
      def _():
        pltpu.sync_copy(pk_hbm.at[pl.ds(cbase + PK_HALF, PK_HALF)],
                        pk_v.at[pl.ds(0, PK_HALF)])

      jrow = j - jnp.where(j + 4 >= PK_HALF, PK_HALF, 0)
      wait_and_scatter(0, 0)
      wait_and_scatter(1, 1)
      drain_scatter(0, 0)

      @pl.when(j + 4 < CHUNKS)
      def _():
        unpack(jrow + 4, 0)
      issue_gather(0, 2)

      drain_scatter(1, 1)

      @pl.when(j + 4 < CHUNKS)
      def _():
        unpack(jrow + 5, 1)
      issue_gather(1, 3)

      wait_and_scatter(0, 2)
      wait_and_scatter(1, 3)
      drain_scatter(0, 2)

      @pl.when(j + 4 < CHUNKS)
      def _():
        unpack(jrow + 6, 2)
        issue_gather(0, 0)

      drain_scatter(1, 3)

      @pl.when(j + 4 < CHUNKS)
      def _():
        unpack(jrow + 7, 3)
        issue_gather(1, 1)

    plsc.subcore_barrier()
    pltpu.sync_copy(acc_x.at[pl.ds(r0, ROWS_PER_TILE)],
                    sx_hbm.at[cid, pl.ds(r0, ROWS_PER_TILE)])

  return k(x, packed2d, zx)


def _sc_scatter_ea(ea128, packed2d, ze):
  @functools.partial(
      pl.kernel,
      mesh=_mesh,
      compiler_params=_sc_params,
      out_type=jax.ShapeDtypeStruct((NC, N_PAD, D_ED), jnp.float32),
      scratch_types=[
          pltpu.VMEM_SHARED((N_PAD, D_ED), jnp.float32),
          pltpu.VMEM((PK_HALF, 128), jnp.int32),
          pltpu.VMEM((4, C), jnp.int32),
          pltpu.VMEM((C // 8, 128), jnp.float32),
          pltpu.VMEM((C // 8, 128), jnp.float32),
          pltpu.VMEM((C, D_ED), jnp.float32),
          pltpu.VMEM((C, D_ED), jnp.float32),
          pltpu.SemaphoreType.DMA,
          pltpu.SemaphoreType.DMA,
          pltpu.SemaphoreType.DMA,
          pltpu.SemaphoreType.DMA,
      ],
  )
  def k(ea_hbm, pk_hbm, ze_hbm, se_hbm,
        acc_e, pk_v, dst_i, eaw0, eaw1, ea0, ea1,
        sl0, sl1, ss0, ss1):
    cid = lax.axis_index("c")
    sid = lax.axis_index("s")
    wid = sid * NC + cid
    cbase = wid * CHUNKS
    eaw = (eaw0, eaw1)
    eab = (ea0, ea1)
    sl = (sl0, sl1)
    ss = (ss0, ss1)

    r0 = sid * ROWS_PER_TILE
    pltpu.sync_copy(ze_hbm, acc_e.at[pl.ds(r0, ROWS_PER_TILE)])
    for rr in range(C):
      ea0[rr, pl.ds(8, L)] = jnp.ones((L,), jnp.float32)
      ea1[rr, pl.ds(8, L)] = jnp.ones((L,), jnp.float32)

    pltpu.sync_copy(pk_hbm.at[pl.ds(cbase, PK_HALF)],
                    pk_v.at[pl.ds(0, PK_HALF)])
    plsc.subcore_barrier()

    def unpack(row, slot):
      for kk in range(C // L):
        pk = pk_v[row, pl.ds(kk * L, L)]
        dst_i[slot, pl.ds(kk * L, L)] = lax.shift_right_logical(pk, 16)

    def ea_row(n):
      g = cbase + n
      return jnp.where(g < N_EDGES // C, g * (C // 8), 0)

    def issue_load(n, p):
      pltpu.async_copy(ea_hbm.at[pl.ds(ea_row(n), C // 8)], eaw[p], sl[p])

    def wait_and_scatter(n, p, slot):
      pltpu.make_async_copy(ea_hbm.at[pl.ds(ea_row(n), C // 8)], eaw[p],
                            sl[p]).wait()
      for r in range(C):
        eab[p][r, pl.ds(0, L)] = eaw[p][r // 8, pl.ds((r % 8) * L, L)]
      pltpu.async_copy(eab[p], acc_e.at[dst_i.at[slot]], ss[p], add=True)

    def drain_scatter(p, slot):
      pltpu.make_async_copy(eab[p], acc_e.at[dst_i.at[slot]], ss[p]).wait()

    for q in range(4):
      unpack(q, q)
    issue_load(0, 0)
    issue_load(1, 1)

    @pl.loop(0, CHUNKS, step=4)
    def _body(j):
      @pl.when(j + 4 == PK_HALF)
      def _():
        pltpu.sync_copy(pk_hbm.at[pl.ds(cbase + PK_HALF, PK_HALF)],
                        pk_v.at[pl.ds(0, PK_HALF)])

      jrow = j - jnp.where(j + 4 >= PK_HALF, PK_HALF, 0)
      wait_and_scatter(j, 0, 0)
      issue_load(j + 2, 0)
      wait_and_scatter(j + 1, 1, 1)
      issue_load(j + 3, 1)
      drain_scatter(0, 0)

      @pl.when(j + 4 < CHUNKS)
      def _():
        unpack(jrow + 4, 0)

      drain_scatter(1, 1)

      @pl.when(j + 4 < CHUNKS)
      def _():
        unpack(jrow + 5, 1)

      wait_and_scatter(j + 2, 0, 2)

      @pl.when(j + 4 < CHUNKS)
      def _():
        issue_load(j + 4, 0)
      wait_and_scatter(j + 3, 1, 3)

      @pl.when(j + 4 < CHUNKS)
      def _():
        issue_load(j + 5, 1)
      drain_scatter(0, 2)

      @pl.when(j + 4 < CHUNKS)
      def _():
        unpack(jrow + 6, 2)

      drain_scatter(1, 3)

      @pl.when(j + 4 < CHUNKS)
      def _():
        unpack(jrow + 7, 3)

    plsc.subcore_barrier()
    pltpu.sync_copy(acc_e.at[pl.ds(r0, ROWS_PER_TILE)],
                    se_hbm.at[cid, pl.ds(r0, ROWS_PER_TILE)])

  return k(ea128, packed2d, ze)


def _tc_body(x_r, sx_r, se_r, w1_r, w2_r, ww1_r, ww2_r, wb_r, out_r):
  sx = sx_r[0] + sx_r[1]
  sed = se_r[0] + se_r[1]
  se = sed[:, 0:D_EDGE]
  invd = 1.0 / jnp.maximum(sed[:, D_EDGE:D_EDGE + 1], 1.0)
  hn = (jnp.dot(sx * invd, w1_r[...], preferred_element_type=jnp.float32)
        + jnp.dot(se * invd, w2_r[...], preferred_element_type=jnp.float32))
  out = (jnp.dot(x_r[...], ww1_r[...], preferred_element_type=jnp.float32)
         + jnp.dot(hn, ww2_r[...], preferred_element_type=jnp.float32)
         + wb_r[...])
  out_r[...] = out


def _tc_combine(x, sums_x, sums_e, w1, w2, ww1, ww2, wb):
  blk = 1000
  grid = N_NODES // blk
  return pl.pallas_call(
      _tc_body,
      grid=(grid,),
      in_specs=[
          pl.BlockSpec((blk, D_FEAT), lambda i: (i, 0)),
          pl.BlockSpec((NC, blk, D_FEAT), lambda i: (0, i, 0)),
          pl.BlockSpec((NC, blk, D_ED), lambda i: (0, i, 0)),
          pl.BlockSpec((D_FEAT, D_OUT), lambda i: (0, 0)),
          pl.BlockSpec((D_EDGE, D_OUT), lambda i: (0, 0)),
          pl.BlockSpec((D_FEAT, D_OUT), lambda i: (0, 0)),
          pl.BlockSpec((D_OUT, D_OUT), lambda i: (0, 0)),
          pl.BlockSpec((1, D_OUT), lambda i: (0, 0)),
      ],
      out_specs=pl.BlockSpec((blk, D_OUT), lambda i: (i, 0)),
      out_shape=jax.ShapeDtypeStruct((N_NODES, D_OUT), jnp.float32),
  )(x, sums_x, sums_e, w1, w2, ww1, ww2, wb)


def kernel(x, edge_index, edge_attr, weight, W_w, W_b, bias):
  src = edge_index[0].astype(jnp.int32)
  dst = edge_index[1].astype(jnp.int32)
  pad = E_PAD - N_EDGES
  fill = jnp.arange(pad, dtype=jnp.int32)
  src_p = jnp.concatenate([src, fill % N_NODES])
  dst_p = jnp.concatenate([dst, N_NODES + fill % (N_PAD - N_NODES)])
  packed = jnp.bitwise_or(jnp.left_shift(dst_p, 16), src_p)
  packed2d = packed.reshape(E_PAD // 128, 128)
  ea128 = edge_attr.reshape(N_EDGES * D_EDGE // 128, 128)
  zx = jnp.zeros((ROWS_PER_TILE // 2, D_FEAT), jnp.float32)
  ze = jnp.zeros((ROWS_PER_TILE, D_ED), jnp.float32)

  sums_x = _sc_gather_scatter_x(x, packed2d, zx)
  ea128_b, sums_x = lax.optimization_barrier((ea128, sums_x))
  sums_e = _sc_scatter_ea(ea128_b, packed2d, ze)

  wb = (W_b + bias).reshape(1, D_OUT)
  return _tc_combine(x, sums_x, sums_e,
                     weight[:D_FEAT], weight[D_FEAT:],
                     W_w[:D_FEAT], W_w[D_FEAT:], wb)
